# Initial kernel scaffold; baseline (speedup 1.0000x reference)
import jax
import jax.numpy as jnp
from jax.experimental import pallas as pl


def kernel(features, edge_index, W, a_src, a_dst):
    raise NotImplementedError("write your pallas kernel here")


# speedup vs baseline: 17.5692x; 17.5692x over previous
import functools

import jax
import jax.numpy as jnp
from jax import lax
from jax.experimental import pallas as pl
from jax.experimental.pallas import tpu as pltpu
from jax.experimental.pallas import tpu_sc as plsc

N = 10000
D = 128
DP = 144
E = 320000
ALPHA = 0.2
NC = 2
NS = 16
NW = NC * NS
EPW = E // NW
B = 80
C = EPW // B
RPT = N // NS


def _pad_h(h):
    return jnp.concatenate(
        [h, jnp.ones((N, 1), jnp.float32), jnp.zeros((N, DP - D - 1), jnp.float32)],
        axis=1)


def _tc_first_body(x_ref, w_ref, a_ref, h_ref, st_ref):
    h = jnp.dot(x_ref[...], w_ref[...], preferred_element_type=jnp.float32)
    h_ref[...] = _pad_h(h)
    st_ref[...] = jnp.dot(h, a_ref[...], preferred_element_type=jnp.float32)


def _tc_first(x, w, a2):
    return pl.pallas_call(
        _tc_first_body,
        out_shape=(
            jax.ShapeDtypeStruct((N, DP), jnp.float32),
            jax.ShapeDtypeStruct((N, 2), jnp.float32),
        ),
    )(x, w, a2)


def _tc_mid_body(parts_ref, w_ref, a_ref, h_ref, st_ref, *, elu):
    acc = parts_ref[0, :, :D] + parts_ref[1, :, :D]
    den = parts_ref[0, :, D] + parts_ref[1, :, D] + 1e-16
    x = acc / den[:, None]
    if elu:
        x = jnp.where(x > 0, x, jnp.exp(x) - 1.0)
    h = jnp.dot(x, w_ref[...], preferred_element_type=jnp.float32)
    h_ref[...] = _pad_h(h)
    st_ref[...] = jnp.dot(h, a_ref[...], preferred_element_type=jnp.float32)


def _tc_mid(parts, w, a2, *, elu):
    return pl.pallas_call(
        functools.partial(_tc_mid_body, elu=elu),
        out_shape=(
            jax.ShapeDtypeStruct((N, DP), jnp.float32),
            jax.ShapeDtypeStruct((N, 2), jnp.float32),
        ),
    )(parts, w, a2)


def _tc_final_body(parts_ref, o_ref):
    acc = parts_ref[0, :, :D] + parts_ref[1, :, :D]
    den = parts_ref[0, :, D] + parts_ref[1, :, D] + 1e-16
    x = acc / den[:, None]
    o_ref[...] = jnp.where(x > 0, x, jnp.exp(x) - 1.0)


def _tc_final(parts):
    return pl.pallas_call(
        _tc_final_body,
        out_shape=jax.ShapeDtypeStruct((N, D), jnp.float32),
    )(parts)


_MESH = plsc.VectorSubcoreMesh(core_axis_name="c", subcore_axis_name="s")


@functools.partial(
    pl.kernel,
    out_type=jax.ShapeDtypeStruct((NC, N, DP), jnp.float32),
    mesh=_MESH,
    compiler_params=pltpu.CompilerParams(
        use_tc_tiling_on_sc=False, needs_layout_passes=False),
    scratch_types=[
        pltpu.VMEM((C, B), jnp.int32),
        pltpu.VMEM((C, B), jnp.int32),
        pltpu.VMEM((B,), jnp.float32),
        pltpu.VMEM((B,), jnp.float32),
        pltpu.VMEM((B,), jnp.float32),
        pltpu.VMEM((B, DP), jnp.float32),
        pltpu.VMEM_SHARED((N, DP), jnp.float32),
        pltpu.SemaphoreType.DMA,
        pltpu.SemaphoreType.DMA,
    ],
)
def _sc_edge(s_hbm, t_hbm, h_hbm, src_hbm, dst_hbm, out_hbm,
             srcb, dstb, sv_b, tv_b, ex_b, rows, out_sh, sem, sem2):
    c = lax.axis_index("c")
    s = lax.axis_index("s")
    wid = c * NS + s

    pltpu.sync_copy(src_hbm.at[wid], srcb)
    pltpu.sync_copy(dst_hbm.at[wid], dstb)

    zero16 = jnp.zeros((16,), jnp.float32)

    def zrows(r, carry):
        for g in range(DP // 16):
            rows[r, pl.ds(g * 16, 16)] = zero16
        return carry

    lax.fori_loop(0, B, zrows, 0)
    for q in range(7):
        pltpu.sync_copy(rows, out_sh.at[pl.ds(s * RPT + q * B, B)])
    pltpu.sync_copy(rows.at[pl.ds(0, RPT - 7 * B)],
                    out_sh.at[pl.ds(s * RPT + 7 * B, RPT - 7 * B)])

    plsc.subcore_barrier()

    def body(j, carry):
        cs = pltpu.async_copy(s_hbm.at[srcb.at[j]], sv_b, sem2)
        ct = pltpu.async_copy(t_hbm.at[dstb.at[j]], tv_b, sem2)
        ch = pltpu.async_copy(h_hbm.at[srcb.at[j]], rows, sem)
        cs.wait()
        ct.wait()
        for g in range(B // 16):
            z = sv_b[pl.ds(g * 16, 16)] + tv_b[pl.ds(g * 16, 16)]
            ex_b[pl.ds(g * 16, 16)] = jnp.exp(jnp.where(z >= 0, z, ALPHA * z))
        ch.wait()
        for b in range(B):
            exs = plsc.load_gather(ex_b, [jnp.full((16,), b, jnp.int32)])
            for dblk in range(DP // 16):
                rows[b, pl.ds(dblk * 16, 16)] = (
                    rows[b, pl.ds(dblk * 16, 16)] * exs)
        pltpu.sync_copy(rows, out_sh.at[dstb.at[j]], add=True)
        return carry

    lax.fori_loop(0, C, body, 0)

    plsc.subcore_barrier()
    pltpu.sync_copy(out_sh.at[pl.ds(s * RPT, RPT)],
                    out_hbm.at[c, pl.ds(s * RPT, RPT)])


def kernel(features, edge_index, W, a_src, a_dst):
    src = edge_index[0].astype(jnp.int32).reshape(NW, C, B)
    dst = edge_index[1].astype(jnp.int32).reshape(NW, C, B)
    a2 = jnp.stack([a_src, a_dst], axis=-1)

    def edge(h, st):
        return _sc_edge(st[:, 0], st[:, 1], h, src, dst)

    h, st = _tc_first(features, W[0], a2[0])
    parts = edge(h, st)
    h, st = _tc_mid(parts, W[1], a2[1], elu=False)
    parts = edge(h, st)
    h, st = _tc_mid(parts, W[2], a2[2], elu=True)
    parts = edge(h, st)
    h, st = _tc_mid(parts, W[3], a2[3], elu=False)
    parts = edge(h, st)
    return _tc_final(parts)


# speedup vs baseline: 26.6421x; 1.5164x over previous
import functools

import jax
import jax.numpy as jnp
from jax import lax
from jax.experimental import pallas as pl
from jax.experimental.pallas import tpu as pltpu
from jax.experimental.pallas import tpu_sc as plsc

N = 10000
D = 128
DP = 144
E = 320000
ALPHA = 0.2
NC = 2
NS = 16
NW = NC * NS
EPW = E // NW
B = 80
C = EPW // B
RPT = N // NS


def _pad_h(h):
    return jnp.concatenate(
        [h, jnp.ones((N, 1), jnp.float32), jnp.zeros((N, DP - D - 1), jnp.float32)],
        axis=1)


def _tc_first_body(x_ref, w_ref, a_ref, h_ref, st_ref):
    h = jnp.dot(x_ref[...], w_ref[...], preferred_element_type=jnp.float32)
    h_ref[...] = _pad_h(h)
    st_ref[...] = jnp.dot(h, a_ref[...], preferred_element_type=jnp.float32)


def _tc_first(x, w, a2):
    return pl.pallas_call(
        _tc_first_body,
        out_shape=(
            jax.ShapeDtypeStruct((N, DP), jnp.float32),
            jax.ShapeDtypeStruct((N, 2), jnp.float32),
        ),
    )(x, w, a2)


def _tc_mid_body(parts_ref, w_ref, a_ref, h_ref, st_ref, *, elu):
    acc = parts_ref[0, :, :D] + parts_ref[1, :, :D]
    den = parts_ref[0, :, D] + parts_ref[1, :, D] + 1e-16
    x = acc / den[:, None]
    if elu:
        x = jnp.where(x > 0, x, jnp.exp(x) - 1.0)
    h = jnp.dot(x, w_ref[...], preferred_element_type=jnp.float32)
    h_ref[...] = _pad_h(h)
    st_ref[...] = jnp.dot(h, a_ref[...], preferred_element_type=jnp.float32)


def _tc_mid(parts, w, a2, *, elu):
    return pl.pallas_call(
        functools.partial(_tc_mid_body, elu=elu),
        out_shape=(
            jax.ShapeDtypeStruct((N, DP), jnp.float32),
            jax.ShapeDtypeStruct((N, 2), jnp.float32),
        ),
    )(parts, w, a2)


def _tc_final_body(parts_ref, o_ref):
    acc = parts_ref[0, :, :D] + parts_ref[1, :, :D]
    den = parts_ref[0, :, D] + parts_ref[1, :, D] + 1e-16
    x = acc / den[:, None]
    o_ref[...] = jnp.where(x > 0, x, jnp.exp(x) - 1.0)


def _tc_final(parts):
    return pl.pallas_call(
        _tc_final_body,
        out_shape=jax.ShapeDtypeStruct((N, D), jnp.float32),
    )(parts)


_MESH = plsc.VectorSubcoreMesh(core_axis_name="c", subcore_axis_name="s")


@functools.partial(
    pl.kernel,
    out_type=jax.ShapeDtypeStruct((NC, N, DP), jnp.float32),
    mesh=_MESH,
    compiler_params=pltpu.CompilerParams(
        use_tc_tiling_on_sc=False, needs_layout_passes=False),
    scratch_types=[
        pltpu.VMEM((2, B), jnp.int32),
        pltpu.VMEM((2, B), jnp.int32),
        pltpu.VMEM((B,), jnp.int32),
        pltpu.VMEM((B,), jnp.int32),
        pltpu.VMEM((B,), jnp.float32),
        pltpu.VMEM((B,), jnp.float32),
        pltpu.VMEM((B,), jnp.float32),
        pltpu.VMEM((B,), jnp.float32),
        pltpu.VMEM((B,), jnp.float32),
        pltpu.VMEM((B,), jnp.float32),
        pltpu.VMEM((B, DP), jnp.float32),
        pltpu.VMEM((B, DP), jnp.float32),
        pltpu.VMEM_SHARED((N, DP), jnp.float32),
        pltpu.SemaphoreType.DMA,
        pltpu.SemaphoreType.DMA,
        pltpu.SemaphoreType.DMA,
        pltpu.SemaphoreType.DMA,
        pltpu.SemaphoreType.DMA,
        pltpu.SemaphoreType.DMA,
        pltpu.SemaphoreType.DMA,
        pltpu.SemaphoreType.DMA,
    ],
)
def _sc_edge(s_hbm, t_hbm, h_hbm, idx_hbm, out_hbm,
             pk0, pk1, dstu0, dstu1, sv0, sv1, tv0, tv1, ex0, ex1,
             rows0, rows1, out_sh,
             six0, six1, sst0, sst1, sh0, sh1, ssc0, ssc1):
    c = lax.axis_index("c")
    s = lax.axis_index("s")
    wid = c * NS + s

    pk = (pk0, pk1)
    dstu = (dstu0, dstu1)
    sv = (sv0, sv1)
    tv = (tv0, tv1)
    ex = (ex0, ex1)
    rows = (rows0, rows1)
    six = (six0, six1)
    sst = (sst0, sst1)
    sh = (sh0, sh1)
    ssc = (ssc0, ssc1)

    zero16 = jnp.zeros((16,), jnp.float32)

    def zrows(r, carry):
        for g in range(DP // 16):
            rows0[r, pl.ds(g * 16, 16)] = zero16
        return carry

    lax.fori_loop(0, B, zrows, 0)
    for q in range(7):
        pltpu.sync_copy(rows0, out_sh.at[pl.ds(s * RPT + q * B, B)])
    pltpu.sync_copy(rows0.at[pl.ds(0, RPT - 7 * B)],
                    out_sh.at[pl.ds(s * RPT + 7 * B, RPT - 7 * B)])

    pltpu.async_copy(idx_hbm.at[wid, 0], pk0, six0).wait()
    pltpu.async_copy(s_hbm.at[pk0.at[0]], sv0, sst0)
    pltpu.async_copy(t_hbm.at[pk0.at[1]], tv0, sst0)
    pltpu.async_copy(h_hbm.at[pk0.at[0]], rows0, sh0)
    pltpu.async_copy(idx_hbm.at[wid, 1], pk1, six1)

    plsc.subcore_barrier()

    def step(j, p, first, pre, pre_idx):
        q = 1 - p
        pltpu.make_async_copy(s_hbm.at[pk[p].at[0]], sv[p], sst[p]).wait()
        pltpu.make_async_copy(t_hbm.at[pk[p].at[1]], tv[p], sst[p]).wait()
        for g in range(B // 16):
            z = sv[p][pl.ds(g * 16, 16)] + tv[p][pl.ds(g * 16, 16)]
            ex[p][pl.ds(g * 16, 16)] = jnp.exp(
                jnp.where(z >= 0, z, ALPHA * z))
            dstu[p][pl.ds(g * 16, 16)] = pk[p][1, pl.ds(g * 16, 16)]
        if not first:
            pltpu.make_async_copy(rows[q], out_sh.at[dstu[q]], ssc[q]).wait()
        if pre:
            pltpu.make_async_copy(idx_hbm.at[wid, 0], pk[q], six[q]).wait()
            pltpu.async_copy(s_hbm.at[pk[q].at[0]], sv[q], sst[q])
            pltpu.async_copy(t_hbm.at[pk[q].at[1]], tv[q], sst[q])
            pltpu.async_copy(h_hbm.at[pk[q].at[0]], rows[q], sh[q])
        pltpu.make_async_copy(h_hbm.at[pk[p].at[0]], rows[p], sh[p]).wait()
        if pre_idx:
            pltpu.async_copy(idx_hbm.at[wid, j + 2], pk[p], six[p])

        def scale_g(g, carry):
            for b16 in range(16):
                r = g * 16 + b16
                exs = plsc.load_gather(
                    ex[p], [jnp.full((16,), r, jnp.int32)])
                for dblk in range(DP // 16):
                    rows[p][r, pl.ds(dblk * 16, 16)] = (
                        rows[p][r, pl.ds(dblk * 16, 16)] * exs)
            return carry

        lax.fori_loop(0, B // 16, scale_g, 0)
        pltpu.async_copy(rows[p], out_sh.at[dstu[p]], ssc[p], add=True)

    step(0, 0, True, True, True)

    def pair(jj, carry):
        step(2 * jj + 1, 1, False, True, True)
        step(2 * jj + 2, 0, False, True, True)
        return carry

    lax.fori_loop(0, (C - 3) // 2, pair, 0)
    step(C - 2, 1, False, True, False)
    step(C - 1, 0, False, False, False)
    pltpu.make_async_copy(rows0, out_sh.at[dstu0], ssc0).wait()

    plsc.subcore_barrier()
    pltpu.sync_copy(out_sh.at[pl.ds(s * RPT, RPT)],
                    out_hbm.at[c, pl.ds(s * RPT, RPT)])


def kernel(features, edge_index, W, a_src, a_dst):
    idx2 = jnp.stack(
        [edge_index[0].astype(jnp.int32).reshape(NW, C, B),
         edge_index[1].astype(jnp.int32).reshape(NW, C, B)],
        axis=2)
    a2 = jnp.stack([a_src, a_dst], axis=-1)

    def edge(h, st):
        return _sc_edge(st[:, 0], st[:, 1], h, idx2)

    h, st = _tc_first(features, W[0], a2[0])
    parts = edge(h, st)
    h, st = _tc_mid(parts, W[1], a2[1], elu=False)
    parts = edge(h, st)
    h, st = _tc_mid(parts, W[2], a2[2], elu=True)
    parts = edge(h, st)
    h, st = _tc_mid(parts, W[3], a2[3], elu=False)
    parts = edge(h, st)
    return _tc_final(parts)


# speedup vs baseline: 28.8163x; 1.0816x over previous
import functools

import jax
import jax.numpy as jnp
from jax import lax
from jax.experimental import pallas as pl
from jax.experimental.pallas import tpu as pltpu
from jax.experimental.pallas import tpu_sc as plsc

N = 10000
D = 128
DP = 144
E = 320000
ALPHA = 0.2
NC = 2
NS = 16
NW = NC * NS
EPW = E // NW
B = 80
C = EPW // B
RPT = N // NS


def _pad_h(h):
    return jnp.concatenate(
        [h, jnp.ones((N, 1), jnp.float32), jnp.zeros((N, DP - D - 1), jnp.float32)],
        axis=1)


def _tc_first_body(x_ref, w_ref, a_ref, h_ref, st_ref):
    h = jnp.dot(x_ref[...], w_ref[...], preferred_element_type=jnp.float32)
    h_ref[...] = _pad_h(h)
    st_ref[...] = jnp.dot(h, a_ref[...], preferred_element_type=jnp.float32)


def _tc_first(x, w, a2):
    return pl.pallas_call(
        _tc_first_body,
        out_shape=(
            jax.ShapeDtypeStruct((N, DP), jnp.float32),
            jax.ShapeDtypeStruct((N, 2), jnp.float32),
        ),
    )(x, w, a2)


def _tc_mid_body(parts_ref, w_ref, a_ref, h_ref, st_ref, *, elu):
    acc = parts_ref[0, :, :D] + parts_ref[1, :, :D]
    den = parts_ref[0, :, D] + parts_ref[1, :, D] + 1e-16
    x = acc / den[:, None]
    if elu:
        x = jnp.where(x > 0, x, jnp.exp(x) - 1.0)
    h = jnp.dot(x, w_ref[...], preferred_element_type=jnp.float32)
    h_ref[...] = _pad_h(h)
    st_ref[...] = jnp.dot(h, a_ref[...], preferred_element_type=jnp.float32)


def _tc_mid(parts, w, a2, *, elu):
    return pl.pallas_call(
        functools.partial(_tc_mid_body, elu=elu),
        out_shape=(
            jax.ShapeDtypeStruct((N, DP), jnp.float32),
            jax.ShapeDtypeStruct((N, 2), jnp.float32),
        ),
    )(parts, w, a2)


def _tc_final_body(parts_ref, o_ref):
    acc = parts_ref[0, :, :D] + parts_ref[1, :, :D]
    den = parts_ref[0, :, D] + parts_ref[1, :, D] + 1e-16
    x = acc / den[:, None]
    o_ref[...] = jnp.where(x > 0, x, jnp.exp(x) - 1.0)


def _tc_final(parts):
    return pl.pallas_call(
        _tc_final_body,
        out_shape=jax.ShapeDtypeStruct((N, D), jnp.float32),
    )(parts)


_MESH = plsc.VectorSubcoreMesh(core_axis_name="c", subcore_axis_name="s")


@functools.partial(
    pl.kernel,
    out_type=jax.ShapeDtypeStruct((NC, N, DP), jnp.float32),
    mesh=_MESH,
    compiler_params=pltpu.CompilerParams(
        use_tc_tiling_on_sc=False, needs_layout_passes=False),
    scratch_types=[
        pltpu.VMEM((2, B), jnp.int32),
        pltpu.VMEM((2, B), jnp.int32),
        pltpu.VMEM((2, B), jnp.int32),
        pltpu.VMEM((B,), jnp.int32),
        pltpu.VMEM((B,), jnp.int32),
        pltpu.VMEM((B,), jnp.int32),
        pltpu.VMEM((B,), jnp.float32),
        pltpu.VMEM((B,), jnp.float32),
        pltpu.VMEM((B,), jnp.float32),
        pltpu.VMEM((B,), jnp.float32),
        pltpu.VMEM((B,), jnp.float32),
        pltpu.VMEM((B,), jnp.float32),
        pltpu.VMEM((B,), jnp.float32),
        pltpu.VMEM((B,), jnp.float32),
        pltpu.VMEM((B,), jnp.float32),
        pltpu.VMEM((B, DP), jnp.float32),
        pltpu.VMEM((B, DP), jnp.float32),
        pltpu.VMEM((B, DP), jnp.float32),
        pltpu.VMEM_SHARED((N, DP), jnp.float32),
        pltpu.SemaphoreType.DMA,
        pltpu.SemaphoreType.DMA,
        pltpu.SemaphoreType.DMA,
        pltpu.SemaphoreType.DMA,
        pltpu.SemaphoreType.DMA,
        pltpu.SemaphoreType.DMA,
        pltpu.SemaphoreType.DMA,
        pltpu.SemaphoreType.DMA,
        pltpu.SemaphoreType.DMA,
        pltpu.SemaphoreType.DMA,
        pltpu.SemaphoreType.DMA,
        pltpu.SemaphoreType.DMA,
    ],
)
def _sc_edge(s_hbm, t_hbm, h_hbm, idx_hbm, out_hbm,
             pk0, pk1, pk2, dstu0, dstu1, dstu2, sv0, sv1, sv2,
             tv0, tv1, tv2, ex0, ex1, ex2, rows0, rows1, rows2, out_sh,
             six0, six1, six2, sst0, sst1, sst2, sh0, sh1, sh2,
             ssc0, ssc1, ssc2):
    c = lax.axis_index("c")
    s = lax.axis_index("s")
    wid = c * NS + s

    pk = (pk0, pk1, pk2)
    dstu = (dstu0, dstu1, dstu2)
    sv = (sv0, sv1, sv2)
    tv = (tv0, tv1, tv2)
    ex = (ex0, ex1, ex2)
    rows = (rows0, rows1, rows2)
    six = (six0, six1, six2)
    sst = (sst0, sst1, sst2)
    sh = (sh0, sh1, sh2)
    ssc = (ssc0, ssc1, ssc2)

    zero16 = jnp.zeros((16,), jnp.float32)

    def zrows(r, carry):
        for g in range(DP // 16):
            rows0[r, pl.ds(g * 16, 16)] = zero16
        return carry

    lax.fori_loop(0, B, zrows, 0)
    for q in range(7):
        pltpu.sync_copy(rows0, out_sh.at[pl.ds(s * RPT + q * B, B)])
    pltpu.sync_copy(rows0.at[pl.ds(0, RPT - 7 * B)],
                    out_sh.at[pl.ds(s * RPT + 7 * B, RPT - 7 * B)])

    pltpu.async_copy(idx_hbm.at[wid, 0], pk0, six0).wait()
    pltpu.async_copy(s_hbm.at[pk0.at[0]], sv0, sst0)
    pltpu.async_copy(t_hbm.at[pk0.at[1]], tv0, sst0)
    pltpu.async_copy(h_hbm.at[pk0.at[0]], rows0, sh0)
    pltpu.async_copy(idx_hbm.at[wid, 1], pk1, six1)

    plsc.subcore_barrier()

    def step(j, p, wait_sc, pre, pre_idx):
        pn = (p + 1) % 3
        pp = (p + 2) % 3
        pltpu.make_async_copy(s_hbm.at[pk[p].at[0]], sv[p], sst[p]).wait()
        pltpu.make_async_copy(t_hbm.at[pk[p].at[1]], tv[p], sst[p]).wait()
        for g in range(B // 16):
            z = sv[p][pl.ds(g * 16, 16)] + tv[p][pl.ds(g * 16, 16)]
            ex[p][pl.ds(g * 16, 16)] = jnp.exp(
                jnp.where(z >= 0, z, ALPHA * z))
            dstu[p][pl.ds(g * 16, 16)] = pk[p][1, pl.ds(g * 16, 16)]
        if wait_sc:
            pltpu.make_async_copy(rows[pn], out_sh.at[dstu[pn]],
                                  ssc[pn]).wait()
        if pre:
            pltpu.make_async_copy(idx_hbm.at[wid, 0], pk[pn], six[pn]).wait()
            pltpu.async_copy(s_hbm.at[pk[pn].at[0]], sv[pn], sst[pn])
            pltpu.async_copy(t_hbm.at[pk[pn].at[1]], tv[pn], sst[pn])
            pltpu.async_copy(h_hbm.at[pk[pn].at[0]], rows[pn], sh[pn])
        pltpu.make_async_copy(h_hbm.at[pk[p].at[0]], rows[p], sh[p]).wait()
        if pre_idx:
            pltpu.async_copy(idx_hbm.at[wid, j + 2], pk[pp], six[pp])

        def scale_g(g, carry):
            for b16 in range(16):
                r = g * 16 + b16
                exs = plsc.load_gather(
                    ex[p], [jnp.full((16,), r, jnp.int32)])
                for dblk in range(DP // 16):
                    rows[p][r, pl.ds(dblk * 16, 16)] = (
                        rows[p][r, pl.ds(dblk * 16, 16)] * exs)
            return carry

        lax.fori_loop(0, B // 16, scale_g, 0)
        pltpu.async_copy(rows[p], out_sh.at[dstu[p]], ssc[p], add=True)

    step(0, 0, False, True, True)
    step(1, 1, False, True, True)

    def triple(jj, carry):
        step(3 * jj + 2, 2, True, True, True)
        step(3 * jj + 3, 0, True, True, True)
        step(3 * jj + 4, 1, True, True, True)
        return carry

    lax.fori_loop(0, (C - 5) // 3, triple, 0)
    step(C - 3, 2, True, True, True)
    step(C - 2, 0, True, True, False)
    step(C - 1, 1, True, False, False)
    pltpu.make_async_copy(rows0, out_sh.at[dstu0], ssc0).wait()
    pltpu.make_async_copy(rows1, out_sh.at[dstu1], ssc1).wait()

    plsc.subcore_barrier()
    pltpu.sync_copy(out_sh.at[pl.ds(s * RPT, RPT)],
                    out_hbm.at[c, pl.ds(s * RPT, RPT)])


def kernel(features, edge_index, W, a_src, a_dst):
    idx2 = jnp.stack(
        [edge_index[0].astype(jnp.int32).reshape(NW, C, B),
         edge_index[1].astype(jnp.int32).reshape(NW, C, B)],
        axis=2)
    a2 = jnp.stack([a_src, a_dst], axis=-1)

    def edge(h, st):
        return _sc_edge(st[:, 0], st[:, 1], h, idx2)

    h, st = _tc_first(features, W[0], a2[0])
    parts = edge(h, st)
    h, st = _tc_mid(parts, W[1], a2[1], elu=False)
    parts = edge(h, st)
    h, st = _tc_mid(parts, W[2], a2[2], elu=True)
    parts = edge(h, st)
    h, st = _tc_mid(parts, W[3], a2[3], elu=False)
    parts = edge(h, st)
    return _tc_final(parts)


# speedup vs baseline: 29.5500x; 1.0255x over previous
import functools

import jax
import jax.numpy as jnp
from jax import lax
from jax.experimental import pallas as pl
from jax.experimental.pallas import tpu as pltpu
from jax.experimental.pallas import tpu_sc as plsc

N = 10000
D = 128
DP = 144
E = 320000
ALPHA = 0.2
NC = 2
NS = 16
NW = NC * NS
EPW = E // NW
B = 80
C = EPW // B
G = 5
RPT = N // NS


def _pad_h(h, st):
    return jnp.concatenate(
        [h, jnp.ones((N, 1), jnp.float32), st[:, :1],
         jnp.zeros((N, DP - D - 2), jnp.float32)], axis=1)


def _tc_first_body(x_ref, w_ref, a_ref, h_ref, st_ref):
    h = jnp.dot(x_ref[...], w_ref[...], preferred_element_type=jnp.float32)
    st = jnp.dot(h, a_ref[...], preferred_element_type=jnp.float32)
    h_ref[...] = _pad_h(h, st)
    st_ref[...] = st


def _tc_first(x, w, a2):
    return pl.pallas_call(
        _tc_first_body,
        out_shape=(
            jax.ShapeDtypeStruct((N, DP), jnp.float32),
            jax.ShapeDtypeStruct((N, 2), jnp.float32),
        ),
    )(x, w, a2)


def _tc_mid_body(parts_ref, w_ref, a_ref, h_ref, st_ref, *, elu):
    acc = parts_ref[0, :, :D] + parts_ref[1, :, :D]
    den = parts_ref[0, :, D] + parts_ref[1, :, D] + 1e-16
    x = acc / den[:, None]
    if elu:
        x = jnp.where(x > 0, x, jnp.exp(x) - 1.0)
    h = jnp.dot(x, w_ref[...], preferred_element_type=jnp.float32)
    st = jnp.dot(h, a_ref[...], preferred_element_type=jnp.float32)
    h_ref[...] = _pad_h(h, st)
    st_ref[...] = st


def _tc_mid(parts, w, a2, *, elu):
    return pl.pallas_call(
        functools.partial(_tc_mid_body, elu=elu),
        out_shape=(
            jax.ShapeDtypeStruct((N, DP), jnp.float32),
            jax.ShapeDtypeStruct((N, 2), jnp.float32),
        ),
    )(parts, w, a2)


def _tc_final_body(parts_ref, o_ref):
    acc = parts_ref[0, :, :D] + parts_ref[1, :, :D]
    den = parts_ref[0, :, D] + parts_ref[1, :, D] + 1e-16
    x = acc / den[:, None]
    o_ref[...] = jnp.where(x > 0, x, jnp.exp(x) - 1.0)


def _tc_final(parts):
    return pl.pallas_call(
        _tc_final_body,
        out_shape=jax.ShapeDtypeStruct((N, D), jnp.float32),
    )(parts)


_MESH = plsc.VectorSubcoreMesh(core_axis_name="c", subcore_axis_name="s")


@functools.partial(
    pl.kernel,
    out_type=jax.ShapeDtypeStruct((NC, N, DP), jnp.float32),
    mesh=_MESH,
    compiler_params=pltpu.CompilerParams(
        use_tc_tiling_on_sc=False, needs_layout_passes=False),
    scratch_types=[
        pltpu.VMEM((3, G, 2, B), jnp.int32),
        pltpu.VMEM((B,), jnp.int32),
        pltpu.VMEM((B,), jnp.int32),
        pltpu.VMEM((B,), jnp.int32),
        pltpu.VMEM((B,), jnp.float32),
        pltpu.VMEM((B,), jnp.float32),
        pltpu.VMEM((B,), jnp.float32),
        pltpu.VMEM((B,), jnp.float32),
        pltpu.VMEM((B,), jnp.float32),
        pltpu.VMEM((B,), jnp.float32),
        pltpu.VMEM((B, DP), jnp.float32),
        pltpu.VMEM((B, DP), jnp.float32),
        pltpu.VMEM((B, DP), jnp.float32),
        pltpu.VMEM_SHARED((N, DP), jnp.float32),
        pltpu.SemaphoreType.DMA,
        pltpu.SemaphoreType.DMA,
        pltpu.SemaphoreType.DMA,
        pltpu.SemaphoreType.DMA,
        pltpu.SemaphoreType.DMA,
        pltpu.SemaphoreType.DMA,
        pltpu.SemaphoreType.DMA,
        pltpu.SemaphoreType.DMA,
        pltpu.SemaphoreType.DMA,
        pltpu.SemaphoreType.DMA,
    ],
)
def _sc_edge(t_hbm, h_hbm, idx_hbm, out_hbm,
             pkg, dstu0, dstu1, dstu2, tv0, tv1, tv2, ex0, ex1, ex2,
             rows0, rows1, rows2, out_sh,
             sgx, sst0, sst1, sst2, sh0, sh1, sh2, ssc0, ssc1, ssc2):
    c = lax.axis_index("c")
    s = lax.axis_index("s")
    wid = c * NS + s

    dstu = (dstu0, dstu1, dstu2)
    tv = (tv0, tv1, tv2)
    ex = (ex0, ex1, ex2)
    rows = (rows0, rows1, rows2)
    sst = (sst0, sst1, sst2)
    sh = (sh0, sh1, sh2)
    ssc = (ssc0, ssc1, ssc2)

    zero16 = jnp.zeros((16,), jnp.float32)

    def zrows(r, carry):
        for g in range(DP // 16):
            rows0[r, pl.ds(g * 16, 16)] = zero16
        return carry

    lax.fori_loop(0, B, zrows, 0)
    for q in range(7):
        pltpu.sync_copy(rows0, out_sh.at[pl.ds(s * RPT + q * B, B)])
    pltpu.sync_copy(rows0.at[pl.ds(0, RPT - 7 * B)],
                    out_sh.at[pl.ds(s * RPT + 7 * B, RPT - 7 * B)])

    pltpu.async_copy(idx_hbm.at[wid, 0], pkg.at[0], sgx).wait()
    pltpu.async_copy(t_hbm.at[pkg.at[0, 0, 1]], tv0, sst0)
    pltpu.async_copy(h_hbm.at[pkg.at[0, 0, 0]], rows0, sh0)
    pltpu.async_copy(idx_hbm.at[wid, 1], pkg.at[1], sgx)

    plsc.subcore_barrier()

    def when(cond, fn):
        if isinstance(cond, bool):
            if cond:
                fn()
        else:
            pl.when(cond)(fn)

    def step(j, p, wait_sc, pre):
        pn = (p + 1) % 3
        gsel = (j // G) % 3
        ksel = j % G
        pltpu.make_async_copy(t_hbm.at[dstu[p]], tv[p], sst[p]).wait()
        for g in range(B // 16):
            dstu[p][pl.ds(g * 16, 16)] = pkg[gsel, ksel, 1, pl.ds(g * 16, 16)]
        if wait_sc:
            pltpu.make_async_copy(rows[pn], out_sh.at[dstu[pn]],
                                  ssc[pn]).wait()
        if isinstance(j, int):
            fetch_cond = j % G == 0 and j <= C - 3 * G
            wait_cond = j % G == G - 1 and j < C - 1
        else:
            fetch_cond = jnp.logical_and(j % G == 0, j <= C - 3 * G)
            wait_cond = jnp.logical_and(j % G == G - 1, j < C - 1)
        def _fetch():
            pltpu.async_copy(
                idx_hbm.at[wid, j // G + 2], pkg.at[(j // G + 2) % 3], sgx)

        when(fetch_cond, _fetch)
        def _gwait():
            pltpu.make_async_copy(idx_hbm.at[wid, 0], pkg.at[0], sgx).wait()

        when(wait_cond, _gwait)
        if pre:
            jn = j + 1
            gn = (jn // G) % 3
            kn = jn % G
            pltpu.async_copy(t_hbm.at[pkg.at[gn, kn, 1]], tv[pn], sst[pn])
            pltpu.async_copy(h_hbm.at[pkg.at[gn, kn, 0]], rows[pn], sh[pn])
        pltpu.make_async_copy(h_hbm.at[dstu[p]], rows[p], sh[p]).wait()
        for g in range(B // 16):
            ridx = lax.iota(jnp.int32, 16) + g * 16
            sv = plsc.load_gather(rows[p], [ridx, jnp.full((16,), D + 1,
                                                           jnp.int32)])
            z = sv + tv[p][pl.ds(g * 16, 16)]
            ex[p][pl.ds(g * 16, 16)] = jnp.exp(
                jnp.where(z >= 0, z, ALPHA * z))

        def scale_g(g, carry):
            for b16 in range(16):
                r = g * 16 + b16
                exs = plsc.load_gather(
                    ex[p], [jnp.full((16,), r, jnp.int32)])
                for dblk in range(DP // 16):
                    rows[p][r, pl.ds(dblk * 16, 16)] = (
                        rows[p][r, pl.ds(dblk * 16, 16)] * exs)
            return carry

        lax.fori_loop(0, B // 16, scale_g, 0)
        pltpu.async_copy(rows[p], out_sh.at[dstu[p]], ssc[p], add=True)

    step(0, 0, False, True)
    step(1, 1, False, True)

    def triple(jj, carry):
        step(3 * jj + 2, 2, True, True)
        step(3 * jj + 3, 0, True, True)
        step(3 * jj + 4, 1, True, True)
        return carry

    lax.fori_loop(0, (C - 5) // 3, triple, 0)
    step(C - 3, 2, True, True)
    step(C - 2, 0, True, True)
    step(C - 1, 1, True, False)
    pltpu.make_async_copy(rows0, out_sh.at[dstu0], ssc0).wait()
    pltpu.make_async_copy(rows1, out_sh.at[dstu1], ssc1).wait()

    plsc.subcore_barrier()
    pltpu.sync_copy(out_sh.at[pl.ds(s * RPT, RPT)],
                    out_hbm.at[c, pl.ds(s * RPT, RPT)])


def kernel(features, edge_index, W, a_src, a_dst):
    idx2 = jnp.stack(
        [edge_index[0].astype(jnp.int32).reshape(NW, C, B),
         edge_index[1].astype(jnp.int32).reshape(NW, C, B)],
        axis=2).reshape(NW, C // G, G, 2, B)
    a2 = jnp.stack([a_src, a_dst], axis=-1)

    def edge(h, st):
        return _sc_edge(st[:, 1], h, idx2)

    h, st = _tc_first(features, W[0], a2[0])
    parts = edge(h, st)
    h, st = _tc_mid(parts, W[1], a2[1], elu=False)
    parts = edge(h, st)
    h, st = _tc_mid(parts, W[2], a2[2], elu=True)
    parts = edge(h, st)
    h, st = _tc_mid(parts, W[3], a2[3], elu=False)
    parts = edge(h, st)
    return _tc_final(parts)
